# baseline (speedup 1.0000x reference)
import numpy as np
import jax
import jax.numpy as jnp
from jax import lax
from jax.experimental import pallas as pl
from jax.experimental.pallas import tpu as pltpu
from jax.experimental.pallas import tpu_sc as plsc

EMB = 16
L = 512
B = 8
NWORKERS = 32
ROWS_PER_W = (B * L) // NWORKERS
WPB = L // ROWS_PER_W

_LO = [0, 1, 2, 3, 4, 5, 7, 10, 14, 20, 28, 41, 59, 85, 123, 177]
_TBL = np.zeros((L,), dtype=np.int32)
for _k, _lo in enumerate(_LO):
    _TBL[_lo:] = _k


def _sc_body(lcas_hbm, lcas_t_hbm, tbl_hbm, out_hbm,
             tbl, lrow, trow, rb0, rb1, kb0, kb1, sem0, sem1):
    c = lax.axis_index("c")
    s = lax.axis_index("s")
    w = s * 2 + c
    b = w // WPB
    i0 = (w % WPB) * ROWS_PER_W

    pltpu.sync_copy(tbl_hbm, tbl)

    zeros16 = jnp.zeros((16,), jnp.float32)
    ones16 = jnp.ones((16,), jnp.float32)
    iota = lax.broadcasted_iota(jnp.int32, (16,), 0)

    def zfull(r, carry):
        rb0[r, pl.ds(0, 16)] = zeros16
        rb0[r, pl.ds(16, 16)] = zeros16
        rb1[r, pl.ds(0, 16)] = zeros16
        rb1[r, pl.ds(16, 16)] = zeros16
        return carry
    lax.fori_loop(0, L, zfull, 0)

    def do_row(rr, rb, kb, sem):
        row_i = i0 + rr
        pltpu.sync_copy(lcas_hbm.at[b, row_i], lrow)
        pltpu.sync_copy(lcas_t_hbm.at[b, row_i], trow)
        ivec = jnp.full((16,), row_i, jnp.int32)

        def chunk(cx, carry):
            o = cx * 16
            jvec = iota + o
            r = lrow[pl.ds(o, 16)]
            t = trow[pl.ds(o, 16)]
            d_r = jnp.where(r < 0, L - 1, jnp.minimum(jnp.abs(r - jvec), L - 1))
            d_l = jnp.where(t < 0, L - 1, jnp.minimum(jnp.abs(t - ivec), L - 1))
            kr = plsc.load_gather(tbl, [d_r]) + EMB
            kl = plsc.load_gather(tbl, [d_l])
            plsc.store_scatter(rb, [jvec, kl], ones16)
            plsc.store_scatter(rb, [jvec, kr], ones16)
            kb[pl.ds(o, 16)] = kl
            kb[pl.ds(L + o, 16)] = kr
            return carry
        lax.fori_loop(0, L // 16, chunk, 0)
        pltpu.make_async_copy(rb, out_hbm.at[b, row_i], sem).start()

    def zero_touched(rb, kb):
        def zc(cx, carry):
            o = cx * 16
            jvec = iota + o
            plsc.store_scatter(rb, [jvec, kb[pl.ds(o, 16)]], zeros16)
            plsc.store_scatter(rb, [jvec, kb[pl.ds(L + o, 16)]], zeros16)
            return carry
        lax.fori_loop(0, L // 16, zc, 0)

    def wait_out(sem):
        pltpu.make_async_copy(rb0, out_hbm.at[b, i0], sem).wait()

    do_row(0, rb0, kb0, sem0)
    do_row(1, rb1, kb1, sem1)

    def pair(pp, carry):
        rr = pp * 2 + 2
        wait_out(sem0)
        zero_touched(rb0, kb0)
        do_row(rr, rb0, kb0, sem0)
        wait_out(sem1)
        zero_touched(rb1, kb1)
        do_row(rr + 1, rb1, kb1, sem1)
        return carry
    lax.fori_loop(0, ROWS_PER_W // 2 - 1, pair, 0)

    wait_out(sem0)
    wait_out(sem1)


def kernel(lcas, eye):
    del eye
    lcas_t = jnp.swapaxes(lcas, 1, 2)
    mesh = plsc.VectorSubcoreMesh(core_axis_name="c", subcore_axis_name="s")
    run = pl.kernel(
        _sc_body,
        out_type=jax.ShapeDtypeStruct((B, L, L, 2 * EMB), jnp.float32),
        mesh=mesh,
        compiler_params=pltpu.CompilerParams(
            needs_layout_passes=False, use_tc_tiling_on_sc=False),
        scratch_types=[
            pltpu.VMEM((L,), jnp.int32),
            pltpu.VMEM((L,), jnp.int32),
            pltpu.VMEM((L,), jnp.int32),
            pltpu.VMEM((L, 2 * EMB), jnp.float32),
            pltpu.VMEM((L, 2 * EMB), jnp.float32),
            pltpu.VMEM((2 * L,), jnp.int32),
            pltpu.VMEM((2 * L,), jnp.int32),
            pltpu.SemaphoreType.DMA,
            pltpu.SemaphoreType.DMA,
        ],
    )
    return run(lcas, lcas_t, jnp.asarray(_TBL))


# speedup vs baseline: 1.1186x; 1.1186x over previous
import numpy as np
import jax
import jax.numpy as jnp
from jax import lax
from jax.experimental import pallas as pl
from jax.experimental.pallas import tpu as pltpu
from jax.experimental.pallas import tpu_sc as plsc

EMB = 16
L = 512
B = 8
NWORKERS = 32
ROWS_PER_W = (B * L) // NWORKERS
WPB = L // ROWS_PER_W

_LO = [0, 1, 2, 3, 4, 5, 7, 10, 14, 20, 28, 41, 59, 85, 123, 177]
_TBL = np.zeros((L,), dtype=np.int32)
for _k, _lo in enumerate(_LO):
    _TBL[_lo:] = _k


_GR = 16
_UNROLL = 8


def _sc_body(lcas_hbm, lcas_t_hbm, tbl_hbm, out_hbm,
             tbl, lrows, trows, rb0, rb1, kb0, kb1, sem0, sem1):
    c = lax.axis_index("c")
    s = lax.axis_index("s")
    w = s * 2 + c
    b = w // WPB
    i0 = (w % WPB) * ROWS_PER_W

    pltpu.sync_copy(tbl_hbm, tbl)

    zeros16 = jnp.zeros((16,), jnp.float32)
    ones16 = jnp.ones((16,), jnp.float32)
    iota = lax.broadcasted_iota(jnp.int32, (16,), 0)

    def zfull(r, carry):
        rb0[r, pl.ds(0, 16)] = zeros16
        rb0[r, pl.ds(16, 16)] = zeros16
        rb1[r, pl.ds(0, 16)] = zeros16
        rb1[r, pl.ds(16, 16)] = zeros16
        return carry
    lax.fori_loop(0, L, zfull, 0)

    def do_row(rloc, row_i, rb, kb, sem):
        ivec = jnp.full((16,), row_i, jnp.int32)

        def chunks(cx, carry):
            for u in range(_UNROLL):
                o = cx * (_UNROLL * 16) + u * 16
                jvec = iota + o
                r = lrows[rloc, pl.ds(o, 16)]
                t = trows[rloc, pl.ds(o, 16)]
                d_r = jnp.where(r < 0, L - 1,
                                jnp.minimum(jnp.abs(r - jvec), L - 1))
                d_l = jnp.where(t < 0, L - 1,
                                jnp.minimum(jnp.abs(t - ivec), L - 1))
                kr = plsc.load_gather(tbl, [d_r]) + EMB
                kl = plsc.load_gather(tbl, [d_l])
                plsc.store_scatter(rb, [jvec, kl], ones16)
                plsc.store_scatter(rb, [jvec, kr], ones16)
                kb[pl.ds(o, 16)] = kl
                kb[pl.ds(L + o, 16)] = kr
            return carry
        lax.fori_loop(0, L // (16 * _UNROLL), chunks, 0)
        pltpu.make_async_copy(rb, out_hbm.at[b, row_i], sem).start()

    def zero_touched(rb, kb):
        def zc(cx, carry):
            for u in range(_UNROLL):
                o = cx * (_UNROLL * 16) + u * 16
                jvec = iota + o
                plsc.store_scatter(rb, [jvec, kb[pl.ds(o, 16)]], zeros16)
                plsc.store_scatter(rb, [jvec, kb[pl.ds(L + o, 16)]], zeros16)
            return carry
        lax.fori_loop(0, L // (16 * _UNROLL), zc, 0)

    def wait_out(sem):
        pltpu.make_async_copy(rb0, out_hbm.at[b, i0], sem).wait()

    def group(g, carry):
        r0 = i0 + g * _GR
        pltpu.sync_copy(lcas_hbm.at[b, pl.ds(r0, _GR)], lrows)
        pltpu.sync_copy(lcas_t_hbm.at[b, pl.ds(r0, _GR)], trows)

        def pair(p, carry2):
            done = g * _GR + p * 2
            for sl, (rb, kb, sem) in enumerate(
                    ((rb0, kb0, sem0), (rb1, kb1, sem1))):
                @pl.when(done >= 2)
                def _():
                    wait_out(sem)
                    zero_touched(rb, kb)
                do_row(p * 2 + sl, r0 + p * 2 + sl, rb, kb, sem)
            return carry2
        lax.fori_loop(0, _GR // 2, pair, 0)
        return carry
    lax.fori_loop(0, ROWS_PER_W // _GR, group, 0)

    wait_out(sem0)
    wait_out(sem1)


def kernel(lcas, eye):
    del eye
    lcas_t = jnp.swapaxes(lcas, 1, 2)
    mesh = plsc.VectorSubcoreMesh(core_axis_name="c", subcore_axis_name="s")
    run = pl.kernel(
        _sc_body,
        out_type=jax.ShapeDtypeStruct((B, L, L, 2 * EMB), jnp.float32),
        mesh=mesh,
        compiler_params=pltpu.CompilerParams(
            needs_layout_passes=False, use_tc_tiling_on_sc=False),
        scratch_types=[
            pltpu.VMEM((L,), jnp.int32),
            pltpu.VMEM((_GR, L), jnp.int32),
            pltpu.VMEM((_GR, L), jnp.int32),
            pltpu.VMEM((L, 2 * EMB), jnp.float32),
            pltpu.VMEM((L, 2 * EMB), jnp.float32),
            pltpu.VMEM((2 * L,), jnp.int32),
            pltpu.VMEM((2 * L,), jnp.int32),
            pltpu.SemaphoreType.DMA,
            pltpu.SemaphoreType.DMA,
        ],
    )
    return run(lcas, lcas_t, jnp.asarray(_TBL))
